# baseline (speedup 1.0000x reference)
import functools

import jax
import jax.numpy as jnp
from jax import lax
from jax.experimental import pallas as pl
from jax.experimental.pallas import tpu as pltpu
from jax.experimental.pallas import tpu_sc as plsc

N = 10000
E = 320000
FD = 128
FDP = 64
MROW = 128
NB = 2000
EB = 2000
EPT = E // 16
GCH = 128
NFULL = EPT // GCH
GTAIL = EPT - NFULL * GCH
MCH = 2000
SCH = 80
SNF = EPT // SCH
NPAD = 10240
NPT = NPAD // 16
NHALF = N // 2
NBF = 1000

_f32 = jnp.float32

_SC_MESH = plsc.VectorSubcoreMesh(core_axis_name="c", subcore_axis_name="s")


def _leaky(x):
    return jnp.where(x >= 0, x, 0.2 * x)


def _proj_body(nf_ref, w_ref, o0, o1, o2, o3):
    p = jnp.dot(nf_ref[...], w_ref[...], preferred_element_type=_f32)
    o0[...] = p[:, 0:128]
    o1[...] = p[:, 128:256]
    o2[...] = p[:, 256:384]
    o3[...] = p[:, 384:512]


def _proj(nf, wp):
    return pl.pallas_call(
        _proj_body,
        grid=(N // NB,),
        in_specs=[
            pl.BlockSpec((NB, FD), lambda i: (i, 0)),
            pl.BlockSpec((FD, 512), lambda i: (0, 0)),
        ],
        out_specs=[pl.BlockSpec((NB, FD), lambda i: (i, 0))] * 4,
        out_shape=[jax.ShapeDtypeStruct((N, FD), _f32)] * 4,
    )(nf, wp)


def _gather(pcs, pcd, pns, pnd, csrc, cdst, nsrc, ndst):
    @functools.partial(
        pl.kernel,
        out_type=[jax.ShapeDtypeStruct((E, FD), _f32)] * 4,
        mesh=_SC_MESH,
        scratch_types=[
            pltpu.VMEM((GCH,), jnp.int32),
            pltpu.VMEM((GCH,), jnp.int32),
            pltpu.VMEM((GCH,), jnp.int32),
            pltpu.VMEM((GCH,), jnp.int32),
            pltpu.VMEM((GCH,), jnp.int32),
            pltpu.VMEM((GCH,), jnp.int32),
            pltpu.VMEM((GTAIL,), jnp.int32),
            pltpu.VMEM((GTAIL,), jnp.int32),
            pltpu.VMEM((GCH, FD), _f32),
            pltpu.VMEM((GCH, FD), _f32),
            pltpu.VMEM((GCH, FD), _f32),
            pltpu.VMEM((GCH, FD), _f32),
            pltpu.VMEM((GCH, FD), _f32),
            pltpu.VMEM((GCH, FD), _f32),
            pltpu.SemaphoreType.DMA,
            pltpu.SemaphoreType.DMA,
            pltpu.SemaphoreType.DMA,
            pltpu.SemaphoreType.DMA,
            pltpu.SemaphoreType.DMA,
            pltpu.SemaphoreType.DMA,
        ],
    )
    def k(pcs_h, pcd_h, pns_h, pnd_h, cs_h, cd_h, ns_h, nd_h,
          ocs_h, ocd_h, ons_h, ond_h,
          ia0, ia1, ia2, ib0, ib1, ib2, ta, tb,
          ba0, ba1, ba2, bb0, bb1, bb2,
          sg0, sg1, sg2, sw0, sw1, sw2):
        c = lax.axis_index("c")
        s = lax.axis_index("s")
        ias = (ia0, ia1, ia2)
        ibs = (ib0, ib1, ib2)
        bas = (ba0, ba1, ba2)
        bbs = (bb0, bb1, bb2)
        sg = (sg0, sg1, sg2)
        sw = (sw0, sw1, sw2)

        def run(src_h, dst_h, tsrc_h, tdst_h, osrc_h, odst_h):
            def lsync(i, b):
                base = s * EPT + i * GCH
                pltpu.sync_copy(src_h.at[pl.ds(base, GCH)], ias[b])
                pltpu.sync_copy(dst_h.at[pl.ds(base, GCH)], ibs[b])

            def gissue(b):
                pltpu.async_copy(tsrc_h.at[ias[b]], bas[b], sg[b])
                pltpu.async_copy(tdst_h.at[ibs[b]], bbs[b], sg[b])

            def gwait(b):
                pltpu.make_async_copy(tsrc_h.at[ias[b]], bas[b], sg[b]).wait()
                pltpu.make_async_copy(tdst_h.at[ibs[b]], bbs[b], sg[b]).wait()

            def wissue(i, b):
                base = s * EPT + i * GCH
                pltpu.async_copy(bas[b], osrc_h.at[pl.ds(base, GCH)], sw[b])
                pltpu.async_copy(bbs[b], odst_h.at[pl.ds(base, GCH)], sw[b])

            def wwait(i, b):
                base = s * EPT + i * GCH
                pltpu.make_async_copy(bas[b], osrc_h.at[pl.ds(base, GCH)],
                                      sw[b]).wait()
                pltpu.make_async_copy(bbs[b], odst_h.at[pl.ds(base, GCH)],
                                      sw[b]).wait()

            for b in range(3):
                lsync(b, b)
                gissue(b)

            def outer(g, _):
                for b in range(3):
                    i = g * 3 + b
                    gwait(b)
                    wissue(i, b)
                    lsync(i + 3, b)
                    wwait(i, b)
                    gissue(b)
                return 0
            lax.fori_loop(0, NFULL // 3 - 1, outer, 0)
            for b in range(3):
                i = NFULL - 3 + b
                gwait(b)
                wissue(i, b)
                wwait(i, b)
            base = s * EPT + NFULL * GCH
            pltpu.sync_copy(src_h.at[pl.ds(base, GTAIL)], ta)
            pltpu.sync_copy(dst_h.at[pl.ds(base, GTAIL)], tb)
            da = pltpu.async_copy(tsrc_h.at[ta], bas[0].at[pl.ds(0, GTAIL)], sg[0])
            db = pltpu.async_copy(tdst_h.at[tb], bbs[0].at[pl.ds(0, GTAIL)], sg[0])
            da.wait()
            db.wait()
            pltpu.sync_copy(bas[0].at[pl.ds(0, GTAIL)], osrc_h.at[pl.ds(base, GTAIL)])
            pltpu.sync_copy(bbs[0].at[pl.ds(0, GTAIL)], odst_h.at[pl.ds(base, GTAIL)])

        @pl.when(c == 0)
        def _():
            run(cs_h, cd_h, pcs_h, pcd_h, ocs_h, ocd_h)

        @pl.when(c == 1)
        def _():
            run(ns_h, nd_h, pns_h, pnd_h, ons_h, ond_h)

    return k(pcs, pcd, pns, pnd, csrc, cdst, nsrc, ndst)


def _cell_body(gs_ref, gd_ref, ef_ref,
               wef_f, b1f, w2f, b2f, w3k, b3k, w3f1, b3f1, w3f2, b3f2,
               wef_b, b1b, w2b, b2b, w3b, b3b,
               out_ref, f2t_ref):
    pre = gs_ref[...] + gd_ref[...]
    x_pre = pre[:, 0:64]
    y_pre = pre[:, 64:128]
    ef = ef_ref[...]
    x = x_pre + jnp.dot(ef[:, 6:10], wef_f[...],
                               preferred_element_type=_f32) + b1f[...]
    xb = _leaky(x).astype(jnp.bfloat16)
    xb = _leaky(jnp.dot(xb, w2f[...].astype(jnp.bfloat16),
                        preferred_element_type=_f32) + b2f[...]).astype(jnp.bfloat16)
    gate = jnp.dot(xb, w3k[...].astype(jnp.bfloat16),
                   preferred_element_type=_f32) + b3k[...]
    gate = 1.0 / (1.0 + jnp.exp(-gate))
    f1 = (jnp.dot(xb, w3f1[...].astype(jnp.bfloat16),
                  preferred_element_type=_f32) + b3f1[...]) * gate
    f2t = lax.dot_general(w3f2[...].astype(jnp.bfloat16), xb,
                          (((0,), (1,)), ((), ())),
                          preferred_element_type=_f32)
    f2t = (f2t + b3f2[...]) * jnp.reshape(gate, (1, EB))
    y = y_pre + jnp.dot(ef[:, 0:6], wef_b[...],
                                 preferred_element_type=_f32) + b1b[...]
    yb = _leaky(y).astype(jnp.bfloat16)
    yb = _leaky(jnp.dot(yb, w2b[...].astype(jnp.bfloat16),
                        preferred_element_type=_f32) + b2b[...]).astype(jnp.bfloat16)
    msgb = jnp.dot(yb, w3b[...].astype(jnp.bfloat16),
                   preferred_element_type=_f32) + b3b[...]
    ones = jnp.ones((EB, 1), _f32)
    pad = jnp.zeros((EB, MROW - 97), _f32)
    out_ref[...] = jnp.concatenate([f1, msgb, ones, pad], axis=1)
    f2t_ref[...] = jnp.reshape(f2t, (1, 32, EB))


def _cell_mlp(gcs, gcd, cef, w):
    wspecs = [pl.BlockSpec(x.shape, lambda i: (0,) * x.ndim) for x in w]
    return pl.pallas_call(
        _cell_body,
        grid=(E // EB,),
        in_specs=[
            pl.BlockSpec((EB, FD), lambda i: (i, 0)),
            pl.BlockSpec((EB, FD), lambda i: (i, 0)),
            pl.BlockSpec((EB, 10), lambda i: (i, 0)),
        ] + wspecs,
        out_specs=[
            pl.BlockSpec((EB, MROW), lambda i: (i, 0)),
            pl.BlockSpec((1, 32, EB), lambda i: (i, 0, 0)),
        ],
        out_shape=[
            jax.ShapeDtypeStruct((E, MROW), _f32),
            jax.ShapeDtypeStruct((E // EB, 32, EB), _f32),
        ],
    )(gcs, gcd, cef, *w)


def _net_body(gs_ref, gd_ref, ef_ref,
              wef_f, b1f, w2f, b2f, w3f, b3f,
              wef_b, b1b, w2b, b2b, w3b, b3b,
              out_ref):
    pre = gs_ref[...] + gd_ref[...]
    x_pre = pre[:, 0:64]
    y_pre = pre[:, 64:128]
    ef = ef_ref[...]
    x = x_pre + jnp.dot(ef, wef_f[...], preferred_element_type=_f32) + b1f[...]
    xb = _leaky(x).astype(jnp.bfloat16)
    xb = _leaky(jnp.dot(xb, w2f[...].astype(jnp.bfloat16),
                        preferred_element_type=_f32) + b2f[...]).astype(jnp.bfloat16)
    msgf = jnp.dot(xb, w3f[...].astype(jnp.bfloat16),
                   preferred_element_type=_f32) + b3f[...]
    y = y_pre + jnp.dot(ef, wef_b[...], preferred_element_type=_f32) + b1b[...]
    yb = _leaky(y).astype(jnp.bfloat16)
    yb = _leaky(jnp.dot(yb, w2b[...].astype(jnp.bfloat16),
                        preferred_element_type=_f32) + b2b[...]).astype(jnp.bfloat16)
    msgb = jnp.dot(yb, w3b[...].astype(jnp.bfloat16),
                   preferred_element_type=_f32) + b3b[...]
    out_ref[...] = jnp.concatenate([msgf, msgb], axis=1)


def _net_mlp(gns, gnd, nef, w):
    wspecs = [pl.BlockSpec(x.shape, lambda i: (0,) * x.ndim) for x in w]
    return pl.pallas_call(
        _net_body,
        grid=(E // EB,),
        in_specs=[
            pl.BlockSpec((EB, FD), lambda i: (i, 0)),
            pl.BlockSpec((EB, FD), lambda i: (i, 0)),
            pl.BlockSpec((EB, 16), lambda i: (i, 0)),
        ] + wspecs,
        out_specs=pl.BlockSpec((EB, MROW), lambda i: (i, 0)),
        out_shape=jax.ShapeDtypeStruct((E, MROW), _f32),
    )(gns, gnd, nef, *w)


def _scatter(mc, mn, cdst, ndst, f2t):
    @functools.partial(
        pl.kernel,
        out_type=[
            jax.ShapeDtypeStruct((N, MROW), _f32),
            jax.ShapeDtypeStruct((N, MROW), _f32),
            jax.ShapeDtypeStruct((32 * N,), _f32),
        ],
        mesh=_SC_MESH,
        scratch_types=[
            pltpu.VMEM((SCH,), jnp.int32),
            pltpu.VMEM((SCH,), jnp.int32),
            pltpu.VMEM((2, SCH, MROW), _f32),
            pltpu.VMEM((MCH,), _f32),
            pltpu.VMEM((MCH,), _f32),
            pltpu.VMEM((MCH,), jnp.int32),
            pltpu.VMEM((MCH,), jnp.int32),
            pltpu.VMEM((N,), _f32),
            pltpu.MemorySpace.VMEM_SHARED((NPAD, MROW), _f32),
            pltpu.SemaphoreType.DMA,
            pltpu.SemaphoreType.DMA,
            pltpu.SemaphoreType.DMA,
            pltpu.SemaphoreType.DMA,
            pltpu.SemaphoreType.DMA,
            pltpu.SemaphoreType.DMA,
            pltpu.SemaphoreType.DMA,
            pltpu.SemaphoreType.DMA,
        ],
        compiler_params=pltpu.CompilerParams(needs_layout_passes=False),
    )
    def k(mc_h, mn_h, cd_h, nd_h, f2t_h,
          accc_h, accn_h, maxo_h,
          ia0, ia1, msg2, fb0, fb1, dbb0, dbb1, maxacc, acc_sh,
          smm0, smm1, smi0, smi1, smf0, smf1, smd0, smd1):
        c = lax.axis_index("c")
        s = lax.axis_index("s")
        t = s * 2 + c
        smm = (smm0, smm1)
        smi = (smi0, smi1)
        smf = (smf0, smf1)
        smd = (smd0, smd1)
        ia = (ia0, ia1)
        fb = (fb0, fb1)
        dbb = (dbb0, dbb1)

        def zrow(r, _):
            for j in range(MROW // 16):
                msg2[0, r, pl.ds(j * 16, 16)] = jnp.zeros((16,), _f32)
            return 0
        lax.fori_loop(0, SCH, zrow, 0)
        for j in range(NPT // SCH):
            pltpu.sync_copy(msg2.at[0], acc_sh.at[pl.ds(s * NPT + j * SCH, SCH)])

        plsc.subcore_barrier()

        def sums(m_h, d_h):
            def ld(i, p):
                base = s * EPT + i * SCH
                pltpu.async_copy(m_h.at[pl.ds(base, SCH)], msg2.at[p], smm[p])
                pltpu.async_copy(d_h.at[pl.ds(base, SCH)], ia[p], smi[p])

            def wait_ld(i, p):
                base = s * EPT + i * SCH
                pltpu.make_async_copy(m_h.at[pl.ds(base, SCH)], msg2.at[p],
                                      smm[p]).wait()
                pltpu.make_async_copy(d_h.at[pl.ds(base, SCH)], ia[p],
                                      smi[p]).wait()

            ld(0, 0)
            ld(1, 1)

            def outer(o, _):
                for p in range(2):
                    i = 2 * o + p
                    wait_ld(i, p)
                    pltpu.sync_copy(msg2.at[p], acc_sh.at[ia[p]], add=True)
                    ld(i + 2, p)
                return 0
            lax.fori_loop(0, (SNF - 2) // 2, outer, 0)
            for p in range(2):
                i = SNF - 2 + p
                wait_ld(i, p)
                pltpu.sync_copy(msg2.at[p], acc_sh.at[ia[p]], add=True)

        @pl.when(c == 0)
        def _():
            sums(mc_h, cd_h)
            def zc(r, _):
                for j in range(MROW // 16):
                    col = jnp.where(lax.iota(jnp.int32, 16) == 1, 1.0, 0.0) \
                        if j == 6 else jnp.zeros((16,), _f32)
                    msg2[0, r, pl.ds(j * 16, 16)] = col
                return 0
            lax.fori_loop(0, SCH, zc, 0)

            def dld(i, p):
                pltpu.async_copy(nd_h.at[pl.ds(s * EPT + i * SCH, SCH)],
                                 ia[p], smi[p])

            def dwait(i, p):
                pltpu.make_async_copy(nd_h.at[pl.ds(s * EPT + i * SCH, SCH)],
                                      ia[p], smi[p]).wait()

            dld(0, 0)
            dld(1, 1)

            def douter(o, _):
                for p in range(2):
                    i = 2 * o + p
                    dwait(i, p)
                    pltpu.sync_copy(msg2.at[0], acc_sh.at[ia[p]], add=True)
                    dld(i + 2, p)
                return 0
            lax.fori_loop(0, (SNF - 2) // 2, douter, 0)
            for p in range(2):
                i = SNF - 2 + p
                dwait(i, p)
                pltpu.sync_copy(msg2.at[0], acc_sh.at[ia[p]], add=True)

        @pl.when(c == 1)
        def _():
            sums(mn_h, nd_h)

        lane = lax.iota(jnp.int32, 16)

        def minit(i, _):
            maxacc[pl.ds(i * 16, 16)] = jnp.full((16,), -jnp.inf, _f32)
            return 0
        lax.fori_loop(0, N // 16, minit, 0)

        def mld(i, p):
            pltpu.async_copy(f2t_h.at[pl.ds(i * (32 * MCH) + t * MCH, MCH)],
                             fb[p], smf[p])
            pltpu.async_copy(cd_h.at[pl.ds(i * MCH, MCH)], dbb[p], smd[p])

        def mwait(i, p):
            pltpu.make_async_copy(f2t_h.at[pl.ds(i * (32 * MCH) + t * MCH, MCH)],
                                  fb[p], smf[p]).wait()
            pltpu.make_async_copy(cd_h.at[pl.ds(i * MCH, MCH)], dbb[p],
                                  smd[p]).wait()

        mld(0, 0)
        mld(1, 1)

        def mchunk(o, _):
            for p in range(2):
                i = 2 * o + p
                mwait(i, p)

                def vec(kk, _):
                    d = dbb[p][pl.ds(kk * 16, 16)]
                    v = fb[p][pl.ds(kk * 16, 16)]
                    ds_, vs = plsc.sort_key_val(d, v)
                    for sft in (1, 2, 4, 8):
                        idxm = jnp.maximum(lane - sft, 0)
                        kp = ds_.at[idxm].get(mode="promise_in_bounds")
                        vp = vs.at[idxm].get(mode="promise_in_bounds")
                        vs = jnp.where(kp == ds_, jnp.maximum(vs, vp), vs)
                    for sft in (1, 2, 4, 8):
                        idxp = jnp.minimum(lane + sft, 15)
                        kp = ds_.at[idxp].get(mode="promise_in_bounds")
                        vp = vs.at[idxp].get(mode="promise_in_bounds")
                        vs = jnp.where(kp == ds_, jnp.maximum(vs, vp), vs)
                    cur = plsc.load_gather(maxacc, [ds_])
                    plsc.store_scatter(maxacc, [ds_], jnp.maximum(cur, vs))
                    return 0

                lax.fori_loop(0, MCH // 16, vec, 0)

                @pl.when(i < E // MCH - 2)
                def _():
                    mld(i + 2, p)
            return 0

        lax.fori_loop(0, E // MCH // 2, mchunk, 0)
        for j in range(N // NBF):
            pltpu.sync_copy(maxacc.at[pl.ds(j * NBF, NBF)],
                            maxo_h.at[pl.ds(j * (32 * NBF) + t * NBF, NBF)])

        plsc.subcore_barrier()

        def acc_out(out_h):
            @pl.when(s < 15)
            def _():
                pltpu.sync_copy(acc_sh.at[pl.ds(s * NPT, NPT)],
                                out_h.at[pl.ds(s * NPT, NPT)])

            @pl.when(s == 15)
            def _():
                pltpu.sync_copy(acc_sh.at[pl.ds(15 * NPT, N - 15 * NPT)],
                                out_h.at[pl.ds(15 * NPT, N - 15 * NPT)])

        @pl.when(c == 0)
        def _():
            acc_out(accc_h)

        @pl.when(c == 1)
        def _():
            acc_out(accn_h)

    return k(mc, mn, cdst, ndst, f2t)


def _final_body(nf_ref, accc_ref, accn_ref, maxo_ref,
                r1nf, r1f1, r1f2, r1bm, r1fn, r1bn, b1, w2, b2, w3, b3,
                out_ref):
    accc = accc_ref[...]
    degc = accc[:, 96:97]
    invc = 1.0 / jnp.maximum(degc, 1.0)
    f1s = accc[:, 0:32]
    bmean = accc[:, 32:96] * invc
    accn = accn_ref[...]
    invn = 1.0 / jnp.maximum(accc[:, 97:98], 1.0)
    fnm = accn[:, 0:64] * invn
    bnm = accn[:, 64:128] * invn
    mx = maxo_ref[0]
    zf2 = lax.dot_general(mx, r1f2[...], (((0,), (0,)), ((), ())),
                          preferred_element_type=_f32)
    zf2 = jnp.where(degc > 0, zf2, 0.0)
    z = (jnp.dot(nf_ref[...], r1nf[...], preferred_element_type=_f32)
         + jnp.dot(f1s, r1f1[...], preferred_element_type=_f32)
         + zf2
         + jnp.dot(bmean, r1bm[...], preferred_element_type=_f32)
         + jnp.dot(fnm, r1fn[...], preferred_element_type=_f32)
         + jnp.dot(bnm, r1bn[...], preferred_element_type=_f32)
         + b1[...])
    h = _leaky(z)
    h = _leaky(jnp.dot(h, w2[...], preferred_element_type=_f32) + b2[...])
    out_ref[...] = jnp.dot(h, w3[...], preferred_element_type=_f32) + b3[...]


def _final(nf, accc, accn, maxo, w):
    wspecs = [pl.BlockSpec(x.shape, lambda i: (0,) * x.ndim) for x in w]
    return pl.pallas_call(
        _final_body,
        grid=(N // NBF,),
        in_specs=[
            pl.BlockSpec((NBF, FD), lambda i: (i, 0)),
            pl.BlockSpec((NBF, MROW), lambda i: (i, 0)),
            pl.BlockSpec((NBF, MROW), lambda i: (i, 0)),
            pl.BlockSpec((1, 32, NBF), lambda i: (i, 0, 0)),
        ] + wspecs,
        out_specs=pl.BlockSpec((NBF, FD), lambda i: (i, 0)),
        out_shape=jax.ShapeDtypeStruct((N, FD), _f32),
    )(nf, accc, accn, maxo, *w)


def kernel(nf, cell_edge_index, cell_edge_feat, net_edge_index, net_edge_feat, params):
    csrc = cell_edge_index[0].astype(jnp.int32)
    cdst = cell_edge_index[1].astype(jnp.int32)
    nsrc = net_edge_index[0].astype(jnp.int32)
    ndst = net_edge_index[1].astype(jnp.int32)

    (wfc1, bfc1), (wfc2, bfc2), (wfc3, bfc3) = params['fc']
    (wbc1, bbc1), (wbc2, bbc2), (wbc3, bbc3) = params['bc']
    (wfn1, bfn1), (wfn2, bfn2), (wfn3, bfn3) = params['fn']
    (wbn1, bbn1), (wbn2, bbn2), (wbn3, bbn3) = params['bn']
    (wr1, br1), (wr2, br2), (wr3, br3) = params['red']

    row = lambda b: jnp.reshape(b, (1, -1))
    wp = jnp.concatenate([
        jnp.concatenate([wfc1[0:128], wbc1[0:128]], axis=1),
        jnp.concatenate([wfc1[128:256], wbc1[128:256]], axis=1),
        jnp.concatenate([wfn1[0:128], wbn1[0:128]], axis=1),
        jnp.concatenate([wfn1[128:256], wbn1[128:256]], axis=1),
    ], axis=1)

    pcs, pcd, pns, pnd = _proj(nf, wp)
    gcs, gcd, gns, gnd = _gather(pcs, pcd, pns, pnd, csrc, cdst, nsrc, ndst)

    cell_w = (wfc1[256:260], row(bfc1), wfc2, row(bfc2),
              wfc3[:, 0:1], row(bfc3[0:1]),
              wfc3[:, 1:33], row(bfc3[1:33]),
              wfc3[:, 33:65], jnp.reshape(bfc3[33:65], (32, 1)),
              wbc1[256:262], row(bbc1), wbc2, row(bbc2), wbc3, row(bbc3))
    mc, f2t = _cell_mlp(gcs, gcd, cell_edge_feat, cell_w)

    net_w = (wfn1[256:272], row(bfn1), wfn2, row(bfn2), wfn3, row(bfn3),
             wbn1[256:272], row(bbn1), wbn2, row(bbn2), wbn3, row(bbn3))
    mn = _net_mlp(gns, gnd, net_edge_feat, net_w)

    accc, accn, maxo = _scatter(mc, mn, cdst, ndst,
                                jnp.reshape(f2t, (E // EB * 32 * EB,)))
    maxo = jnp.reshape(maxo, (N // NBF, 32, NBF))

    red_w = (wr1[0:128], wr1[128:160], wr1[160:192], wr1[192:256],
             wr1[256:320], wr1[320:384], row(br1), wr2, row(br2), wr3, row(br3))
    return _final(nf, accc, accn, maxo, red_w)


# speedup vs baseline: 1.0636x; 1.0636x over previous
import functools

import jax
import jax.numpy as jnp
from jax import lax
from jax.experimental import pallas as pl
from jax.experimental.pallas import tpu as pltpu
from jax.experimental.pallas import tpu_sc as plsc

N = 10000
E = 320000
FD = 128
FDP = 64
MROW = 128
NB = 2000
EB = 2000
EPT = E // 16
GCH = 128
NFULL = EPT // GCH
GTAIL = EPT - NFULL * GCH
MCH = 2000
SCH = 80
SNF = EPT // SCH
NPAD = 10240
NPT = NPAD // 16
NHALF = N // 2
NBF = 1000

_f32 = jnp.float32

_SC_MESH = plsc.VectorSubcoreMesh(core_axis_name="c", subcore_axis_name="s")


def _leaky(x):
    return jnp.where(x >= 0, x, 0.2 * x)


def _proj_body(nf_ref, w_ref, o0, o1, o2, o3):
    p = jnp.dot(nf_ref[...], w_ref[...], preferred_element_type=_f32)
    o0[...] = p[:, 0:128]
    o1[...] = p[:, 128:256]
    o2[...] = p[:, 256:384]
    o3[...] = p[:, 384:512]


def _proj(nf, wp):
    return pl.pallas_call(
        _proj_body,
        grid=(N // NB,),
        in_specs=[
            pl.BlockSpec((NB, FD), lambda i: (i, 0)),
            pl.BlockSpec((FD, 512), lambda i: (0, 0)),
        ],
        out_specs=[pl.BlockSpec((NB, FD), lambda i: (i, 0))] * 4,
        out_shape=[jax.ShapeDtypeStruct((N, FD), _f32)] * 4,
    )(nf, wp)


def _gather(pcs, pcd, pns, pnd, csrc, cdst, nsrc, ndst):
    @functools.partial(
        pl.kernel,
        out_type=[jax.ShapeDtypeStruct((E, FD), _f32)] * 4,
        mesh=_SC_MESH,
        scratch_types=[
            pltpu.VMEM((GCH,), jnp.int32),
            pltpu.VMEM((GCH,), jnp.int32),
            pltpu.VMEM((GCH,), jnp.int32),
            pltpu.VMEM((GCH,), jnp.int32),
            pltpu.VMEM((GCH,), jnp.int32),
            pltpu.VMEM((GCH,), jnp.int32),
            pltpu.VMEM((GTAIL,), jnp.int32),
            pltpu.VMEM((GTAIL,), jnp.int32),
            pltpu.VMEM((GCH, FD), _f32),
            pltpu.VMEM((GCH, FD), _f32),
            pltpu.VMEM((GCH, FD), _f32),
            pltpu.VMEM((GCH, FD), _f32),
            pltpu.VMEM((GCH, FD), _f32),
            pltpu.VMEM((GCH, FD), _f32),
            pltpu.SemaphoreType.DMA,
            pltpu.SemaphoreType.DMA,
            pltpu.SemaphoreType.DMA,
            pltpu.SemaphoreType.DMA,
            pltpu.SemaphoreType.DMA,
            pltpu.SemaphoreType.DMA,
        ],
    )
    def k(pcs_h, pcd_h, pns_h, pnd_h, cs_h, cd_h, ns_h, nd_h,
          ocs_h, ocd_h, ons_h, ond_h,
          ia0, ia1, ia2, ib0, ib1, ib2, ta, tb,
          ba0, ba1, ba2, bb0, bb1, bb2,
          sg0, sg1, sg2, sw0, sw1, sw2):
        c = lax.axis_index("c")
        s = lax.axis_index("s")
        ias = (ia0, ia1, ia2)
        ibs = (ib0, ib1, ib2)
        bas = (ba0, ba1, ba2)
        bbs = (bb0, bb1, bb2)
        sg = (sg0, sg1, sg2)
        sw = (sw0, sw1, sw2)

        def run(src_h, dst_h, tsrc_h, tdst_h, osrc_h, odst_h):
            def lsync(i, b):
                base = s * EPT + i * GCH
                pltpu.sync_copy(src_h.at[pl.ds(base, GCH)], ias[b])
                pltpu.sync_copy(dst_h.at[pl.ds(base, GCH)], ibs[b])

            def gissue(b):
                pltpu.async_copy(tsrc_h.at[ias[b]], bas[b], sg[b])
                pltpu.async_copy(tdst_h.at[ibs[b]], bbs[b], sg[b])

            def gwait(b):
                pltpu.make_async_copy(tsrc_h.at[ias[b]], bas[b], sg[b]).wait()
                pltpu.make_async_copy(tdst_h.at[ibs[b]], bbs[b], sg[b]).wait()

            def wissue(i, b):
                base = s * EPT + i * GCH
                pltpu.async_copy(bas[b], osrc_h.at[pl.ds(base, GCH)], sw[b])
                pltpu.async_copy(bbs[b], odst_h.at[pl.ds(base, GCH)], sw[b])

            def wwait(i, b):
                base = s * EPT + i * GCH
                pltpu.make_async_copy(bas[b], osrc_h.at[pl.ds(base, GCH)],
                                      sw[b]).wait()
                pltpu.make_async_copy(bbs[b], odst_h.at[pl.ds(base, GCH)],
                                      sw[b]).wait()

            for b in range(3):
                lsync(b, b)
                gissue(b)

            def outer(g, _):
                for b in range(3):
                    i = g * 3 + b
                    gwait(b)
                    wissue(i, b)
                    lsync(i + 3, b)
                    wwait(i, b)
                    gissue(b)
                return 0
            lax.fori_loop(0, NFULL // 3 - 1, outer, 0)
            for b in range(3):
                i = NFULL - 3 + b
                gwait(b)
                wissue(i, b)
                wwait(i, b)
            base = s * EPT + NFULL * GCH
            pltpu.sync_copy(src_h.at[pl.ds(base, GTAIL)], ta)
            pltpu.sync_copy(dst_h.at[pl.ds(base, GTAIL)], tb)
            da = pltpu.async_copy(tsrc_h.at[ta], bas[0].at[pl.ds(0, GTAIL)], sg[0])
            db = pltpu.async_copy(tdst_h.at[tb], bbs[0].at[pl.ds(0, GTAIL)], sg[0])
            da.wait()
            db.wait()
            pltpu.sync_copy(bas[0].at[pl.ds(0, GTAIL)], osrc_h.at[pl.ds(base, GTAIL)])
            pltpu.sync_copy(bbs[0].at[pl.ds(0, GTAIL)], odst_h.at[pl.ds(base, GTAIL)])

        @pl.when(c == 0)
        def _():
            run(cs_h, cd_h, pcs_h, pcd_h, ocs_h, ocd_h)

        @pl.when(c == 1)
        def _():
            run(ns_h, nd_h, pns_h, pnd_h, ons_h, ond_h)

    return k(pcs, pcd, pns, pnd, csrc, cdst, nsrc, ndst)


def _cell_body(gs_ref, gd_ref, ef_ref,
               wef_f, b1f, w2f, b2f, w3k, b3k, w3f1, b3f1, w3f2, b3f2,
               wef_b, b1b, w2b, b2b, w3b, b3b,
               out_ref, f2t_ref):
    pre = gs_ref[...] + gd_ref[...]
    x_pre = pre[:, 0:64]
    y_pre = pre[:, 64:128]
    ef = ef_ref[...]
    x = x_pre + jnp.dot(ef[:, 6:10], wef_f[...],
                               preferred_element_type=_f32) + b1f[...]
    x = _leaky(x)
    x = _leaky(jnp.dot(x, w2f[...], preferred_element_type=_f32) + b2f[...])
    gate = jnp.dot(x, w3k[...], preferred_element_type=_f32) + b3k[...]
    gate = 1.0 / (1.0 + jnp.exp(-gate))
    f1 = (jnp.dot(x, w3f1[...], preferred_element_type=_f32) + b3f1[...]) * gate
    f2t = lax.dot_general(w3f2[...], x, (((0,), (1,)), ((), ())),
                          preferred_element_type=_f32)
    f2t = (f2t + b3f2[...]) * jnp.reshape(gate, (1, EB))
    y = y_pre + jnp.dot(ef[:, 0:6], wef_b[...],
                                 preferred_element_type=_f32) + b1b[...]
    y = _leaky(y)
    y = _leaky(jnp.dot(y, w2b[...], preferred_element_type=_f32) + b2b[...])
    msgb = jnp.dot(y, w3b[...], preferred_element_type=_f32) + b3b[...]
    ones = jnp.ones((EB, 1), _f32)
    pad = jnp.zeros((EB, MROW - 97), _f32)
    out_ref[...] = jnp.concatenate([f1, msgb, ones, pad], axis=1)
    f2t_ref[...] = jnp.reshape(f2t, (1, 32, EB))


def _cell_mlp(gcs, gcd, cef, w):
    wspecs = [pl.BlockSpec(x.shape, lambda i: (0,) * x.ndim) for x in w]
    return pl.pallas_call(
        _cell_body,
        grid=(E // EB,),
        in_specs=[
            pl.BlockSpec((EB, FD), lambda i: (i, 0)),
            pl.BlockSpec((EB, FD), lambda i: (i, 0)),
            pl.BlockSpec((EB, 10), lambda i: (i, 0)),
        ] + wspecs,
        out_specs=[
            pl.BlockSpec((EB, MROW), lambda i: (i, 0)),
            pl.BlockSpec((1, 32, EB), lambda i: (i, 0, 0)),
        ],
        out_shape=[
            jax.ShapeDtypeStruct((E, MROW), _f32),
            jax.ShapeDtypeStruct((E // EB, 32, EB), _f32),
        ],
    )(gcs, gcd, cef, *w)


def _net_body(gs_ref, gd_ref, ef_ref,
              wef_f, b1f, w2f, b2f, w3f, b3f,
              wef_b, b1b, w2b, b2b, w3b, b3b,
              out_ref):
    pre = gs_ref[...] + gd_ref[...]
    x_pre = pre[:, 0:64]
    y_pre = pre[:, 64:128]
    ef = ef_ref[...]
    x = x_pre + jnp.dot(ef, wef_f[...], preferred_element_type=_f32) + b1f[...]
    x = _leaky(x)
    x = _leaky(jnp.dot(x, w2f[...], preferred_element_type=_f32) + b2f[...])
    msgf = jnp.dot(x, w3f[...], preferred_element_type=_f32) + b3f[...]
    y = y_pre + jnp.dot(ef, wef_b[...], preferred_element_type=_f32) + b1b[...]
    y = _leaky(y)
    y = _leaky(jnp.dot(y, w2b[...], preferred_element_type=_f32) + b2b[...])
    msgb = jnp.dot(y, w3b[...], preferred_element_type=_f32) + b3b[...]
    out_ref[...] = jnp.concatenate([msgf, msgb], axis=1)


def _net_mlp(gns, gnd, nef, w):
    wspecs = [pl.BlockSpec(x.shape, lambda i: (0,) * x.ndim) for x in w]
    return pl.pallas_call(
        _net_body,
        grid=(E // EB,),
        in_specs=[
            pl.BlockSpec((EB, FD), lambda i: (i, 0)),
            pl.BlockSpec((EB, FD), lambda i: (i, 0)),
            pl.BlockSpec((EB, 16), lambda i: (i, 0)),
        ] + wspecs,
        out_specs=pl.BlockSpec((EB, MROW), lambda i: (i, 0)),
        out_shape=jax.ShapeDtypeStruct((E, MROW), _f32),
    )(gns, gnd, nef, *w)


def _scatter(mc, mn, cdst, ndst, f2t):
    @functools.partial(
        pl.kernel,
        out_type=[
            jax.ShapeDtypeStruct((N, MROW), _f32),
            jax.ShapeDtypeStruct((N, MROW), _f32),
            jax.ShapeDtypeStruct((32 * N,), _f32),
        ],
        mesh=_SC_MESH,
        scratch_types=[
            pltpu.VMEM((SCH,), jnp.int32),
            pltpu.VMEM((SCH,), jnp.int32),
            pltpu.VMEM((2, SCH, MROW), _f32),
            pltpu.VMEM((MCH,), _f32),
            pltpu.VMEM((MCH,), _f32),
            pltpu.VMEM((MCH,), jnp.int32),
            pltpu.VMEM((MCH,), jnp.int32),
            pltpu.VMEM((N,), _f32),
            pltpu.MemorySpace.VMEM_SHARED((NPAD, MROW), _f32),
            pltpu.SemaphoreType.DMA,
            pltpu.SemaphoreType.DMA,
            pltpu.SemaphoreType.DMA,
            pltpu.SemaphoreType.DMA,
            pltpu.SemaphoreType.DMA,
            pltpu.SemaphoreType.DMA,
            pltpu.SemaphoreType.DMA,
            pltpu.SemaphoreType.DMA,
        ],
        compiler_params=pltpu.CompilerParams(needs_layout_passes=False),
    )
    def k(mc_h, mn_h, cd_h, nd_h, f2t_h,
          accc_h, accn_h, maxo_h,
          ia0, ia1, msg2, fb0, fb1, dbb0, dbb1, maxacc, acc_sh,
          smm0, smm1, smi0, smi1, smf0, smf1, smd0, smd1):
        c = lax.axis_index("c")
        s = lax.axis_index("s")
        t = s * 2 + c
        smm = (smm0, smm1)
        smi = (smi0, smi1)
        smf = (smf0, smf1)
        smd = (smd0, smd1)
        ia = (ia0, ia1)
        fb = (fb0, fb1)
        dbb = (dbb0, dbb1)

        def zrow(r, _):
            for j in range(MROW // 16):
                msg2[0, r, pl.ds(j * 16, 16)] = jnp.zeros((16,), _f32)
            return 0
        lax.fori_loop(0, SCH, zrow, 0)
        for j in range(NPT // SCH):
            pltpu.sync_copy(msg2.at[0], acc_sh.at[pl.ds(s * NPT + j * SCH, SCH)])

        plsc.subcore_barrier()

        def sums(m_h, d_h):
            def ld(i, p):
                base = s * EPT + i * SCH
                pltpu.async_copy(m_h.at[pl.ds(base, SCH)], msg2.at[p], smm[p])
                pltpu.async_copy(d_h.at[pl.ds(base, SCH)], ia[p], smi[p])

            def wait_ld(i, p):
                base = s * EPT + i * SCH
                pltpu.make_async_copy(m_h.at[pl.ds(base, SCH)], msg2.at[p],
                                      smm[p]).wait()
                pltpu.make_async_copy(d_h.at[pl.ds(base, SCH)], ia[p],
                                      smi[p]).wait()

            ld(0, 0)
            ld(1, 1)

            def outer(o, _):
                for p in range(2):
                    i = 2 * o + p
                    wait_ld(i, p)
                    pltpu.sync_copy(msg2.at[p], acc_sh.at[ia[p]], add=True)
                    ld(i + 2, p)
                return 0
            lax.fori_loop(0, (SNF - 2) // 2, outer, 0)
            for p in range(2):
                i = SNF - 2 + p
                wait_ld(i, p)
                pltpu.sync_copy(msg2.at[p], acc_sh.at[ia[p]], add=True)

        @pl.when(c == 0)
        def _():
            sums(mc_h, cd_h)
            def zc(r, _):
                for j in range(MROW // 16):
                    col = jnp.where(lax.iota(jnp.int32, 16) == 1, 1.0, 0.0) \
                        if j == 6 else jnp.zeros((16,), _f32)
                    msg2[0, r, pl.ds(j * 16, 16)] = col
                return 0
            lax.fori_loop(0, SCH, zc, 0)

            def dld(i, p):
                pltpu.async_copy(nd_h.at[pl.ds(s * EPT + i * SCH, SCH)],
                                 ia[p], smi[p])

            def dwait(i, p):
                pltpu.make_async_copy(nd_h.at[pl.ds(s * EPT + i * SCH, SCH)],
                                      ia[p], smi[p]).wait()

            dld(0, 0)
            dld(1, 1)

            def douter(o, _):
                for p in range(2):
                    i = 2 * o + p
                    dwait(i, p)
                    pltpu.sync_copy(msg2.at[0], acc_sh.at[ia[p]], add=True)
                    dld(i + 2, p)
                return 0
            lax.fori_loop(0, (SNF - 2) // 2, douter, 0)
            for p in range(2):
                i = SNF - 2 + p
                dwait(i, p)
                pltpu.sync_copy(msg2.at[0], acc_sh.at[ia[p]], add=True)

        @pl.when(c == 1)
        def _():
            sums(mn_h, nd_h)

        lane = lax.iota(jnp.int32, 16)

        def minit(i, _):
            maxacc[pl.ds(i * 16, 16)] = jnp.full((16,), -jnp.inf, _f32)
            return 0
        lax.fori_loop(0, N // 16, minit, 0)

        def mld(i, p):
            pltpu.async_copy(f2t_h.at[pl.ds(i * (32 * MCH) + t * MCH, MCH)],
                             fb[p], smf[p])
            pltpu.async_copy(cd_h.at[pl.ds(i * MCH, MCH)], dbb[p], smd[p])

        def mwait(i, p):
            pltpu.make_async_copy(f2t_h.at[pl.ds(i * (32 * MCH) + t * MCH, MCH)],
                                  fb[p], smf[p]).wait()
            pltpu.make_async_copy(cd_h.at[pl.ds(i * MCH, MCH)], dbb[p],
                                  smd[p]).wait()

        mld(0, 0)
        mld(1, 1)

        def mchunk(o, _):
            for p in range(2):
                i = 2 * o + p
                mwait(i, p)

                def vec(kk, _):
                    d = dbb[p][pl.ds(kk * 16, 16)]
                    v = fb[p][pl.ds(kk * 16, 16)]
                    ds_, vs = plsc.sort_key_val(d, v)
                    for sft in (1, 2, 4, 8):
                        idxm = jnp.maximum(lane - sft, 0)
                        kp = ds_.at[idxm].get(mode="promise_in_bounds")
                        vp = vs.at[idxm].get(mode="promise_in_bounds")
                        vs = jnp.where(kp == ds_, jnp.maximum(vs, vp), vs)
                    for sft in (1, 2, 4, 8):
                        idxp = jnp.minimum(lane + sft, 15)
                        kp = ds_.at[idxp].get(mode="promise_in_bounds")
                        vp = vs.at[idxp].get(mode="promise_in_bounds")
                        vs = jnp.where(kp == ds_, jnp.maximum(vs, vp), vs)
                    cur = plsc.load_gather(maxacc, [ds_])
                    plsc.store_scatter(maxacc, [ds_], jnp.maximum(cur, vs))
                    return 0

                lax.fori_loop(0, MCH // 16, vec, 0)

                @pl.when(i < E // MCH - 2)
                def _():
                    mld(i + 2, p)
            return 0

        lax.fori_loop(0, E // MCH // 2, mchunk, 0)
        for j in range(N // NBF):
            pltpu.sync_copy(maxacc.at[pl.ds(j * NBF, NBF)],
                            maxo_h.at[pl.ds(j * (32 * NBF) + t * NBF, NBF)])

        plsc.subcore_barrier()

        def acc_out(out_h):
            @pl.when(s < 15)
            def _():
                pltpu.sync_copy(acc_sh.at[pl.ds(s * NPT, NPT)],
                                out_h.at[pl.ds(s * NPT, NPT)])

            @pl.when(s == 15)
            def _():
                pltpu.sync_copy(acc_sh.at[pl.ds(15 * NPT, N - 15 * NPT)],
                                out_h.at[pl.ds(15 * NPT, N - 15 * NPT)])

        @pl.when(c == 0)
        def _():
            acc_out(accc_h)

        @pl.when(c == 1)
        def _():
            acc_out(accn_h)

    return k(mc, mn, cdst, ndst, f2t)


def _final_body(nf_ref, accc_ref, accn_ref, maxo_ref,
                r1nf, r1f1, r1f2, r1bm, r1fn, r1bn, b1, w2, b2, w3, b3,
                out_ref):
    accc = accc_ref[...]
    degc = accc[:, 96:97]
    invc = 1.0 / jnp.maximum(degc, 1.0)
    f1s = accc[:, 0:32]
    bmean = accc[:, 32:96] * invc
    accn = accn_ref[...]
    invn = 1.0 / jnp.maximum(accc[:, 97:98], 1.0)
    fnm = accn[:, 0:64] * invn
    bnm = accn[:, 64:128] * invn
    mx = maxo_ref[0]
    zf2 = lax.dot_general(mx, r1f2[...], (((0,), (0,)), ((), ())),
                          preferred_element_type=_f32)
    zf2 = jnp.where(degc > 0, zf2, 0.0)
    z = (jnp.dot(nf_ref[...], r1nf[...], preferred_element_type=_f32)
         + jnp.dot(f1s, r1f1[...], preferred_element_type=_f32)
         + zf2
         + jnp.dot(bmean, r1bm[...], preferred_element_type=_f32)
         + jnp.dot(fnm, r1fn[...], preferred_element_type=_f32)
         + jnp.dot(bnm, r1bn[...], preferred_element_type=_f32)
         + b1[...])
    h = _leaky(z)
    h = _leaky(jnp.dot(h, w2[...], preferred_element_type=_f32) + b2[...])
    out_ref[...] = jnp.dot(h, w3[...], preferred_element_type=_f32) + b3[...]


def _final(nf, accc, accn, maxo, w):
    wspecs = [pl.BlockSpec(x.shape, lambda i: (0,) * x.ndim) for x in w]
    return pl.pallas_call(
        _final_body,
        grid=(N // NBF,),
        in_specs=[
            pl.BlockSpec((NBF, FD), lambda i: (i, 0)),
            pl.BlockSpec((NBF, MROW), lambda i: (i, 0)),
            pl.BlockSpec((NBF, MROW), lambda i: (i, 0)),
            pl.BlockSpec((1, 32, NBF), lambda i: (i, 0, 0)),
        ] + wspecs,
        out_specs=pl.BlockSpec((NBF, FD), lambda i: (i, 0)),
        out_shape=jax.ShapeDtypeStruct((N, FD), _f32),
    )(nf, accc, accn, maxo, *w)


def kernel(nf, cell_edge_index, cell_edge_feat, net_edge_index, net_edge_feat, params):
    csrc = cell_edge_index[0].astype(jnp.int32)
    cdst = cell_edge_index[1].astype(jnp.int32)
    nsrc = net_edge_index[0].astype(jnp.int32)
    ndst = net_edge_index[1].astype(jnp.int32)

    (wfc1, bfc1), (wfc2, bfc2), (wfc3, bfc3) = params['fc']
    (wbc1, bbc1), (wbc2, bbc2), (wbc3, bbc3) = params['bc']
    (wfn1, bfn1), (wfn2, bfn2), (wfn3, bfn3) = params['fn']
    (wbn1, bbn1), (wbn2, bbn2), (wbn3, bbn3) = params['bn']
    (wr1, br1), (wr2, br2), (wr3, br3) = params['red']

    row = lambda b: jnp.reshape(b, (1, -1))
    wp = jnp.concatenate([
        jnp.concatenate([wfc1[0:128], wbc1[0:128]], axis=1),
        jnp.concatenate([wfc1[128:256], wbc1[128:256]], axis=1),
        jnp.concatenate([wfn1[0:128], wbn1[0:128]], axis=1),
        jnp.concatenate([wfn1[128:256], wbn1[128:256]], axis=1),
    ], axis=1)

    pcs, pcd, pns, pnd = _proj(nf, wp)
    gcs, gcd, gns, gnd = _gather(pcs, pcd, pns, pnd, csrc, cdst, nsrc, ndst)

    cell_w = (wfc1[256:260], row(bfc1), wfc2, row(bfc2),
              wfc3[:, 0:1], row(bfc3[0:1]),
              wfc3[:, 1:33], row(bfc3[1:33]),
              wfc3[:, 33:65], jnp.reshape(bfc3[33:65], (32, 1)),
              wbc1[256:262], row(bbc1), wbc2, row(bbc2), wbc3, row(bbc3))
    mc, f2t = _cell_mlp(gcs, gcd, cell_edge_feat, cell_w)

    net_w = (wfn1[256:272], row(bfn1), wfn2, row(bfn2), wfn3, row(bfn3),
             wbn1[256:272], row(bbn1), wbn2, row(bbn2), wbn3, row(bbn3))
    mn = _net_mlp(gns, gnd, net_edge_feat, net_w)

    accc, accn, maxo = _scatter(mc, mn, cdst, ndst,
                                jnp.reshape(f2t, (E // EB * 32 * EB,)))
    maxo = jnp.reshape(maxo, (N // NBF, 32, NBF))

    red_w = (wr1[0:128], wr1[128:160], wr1[160:192], wr1[192:256],
             wr1[256:320], wr1[320:384], row(br1), wr2, row(br2), wr3, row(br3))
    return _final(nf, accc, accn, maxo, red_w)
